# baseline (speedup 1.0000x reference)
import functools

import jax
import jax.numpy as jnp
from jax import lax
from jax.experimental import pallas as pl
from jax.experimental.pallas import tpu as pltpu
from jax.experimental.pallas import tpu_sc as plsc

N = 10000
E = 320000
D = 128
H = 128
C = 10
G = 16
EPS = 1e-5

NC = 2
NS = 16
NW = NC * NS
EPW = E // NW
CH = 80
NCH = EPW // CH
NBUF = 5
NP = 10240
RPT = NP // NS
RB = 1000
NB = N // RB


def _sc_mesh():
    return plsc.VectorSubcoreMesh(core_axis_name="c", subcore_axis_name="s")


def _sc_degree(dst3, ones128, z128):
    @functools.partial(
        pl.kernel,
        out_type=jax.ShapeDtypeStruct((NC, NP, H), jnp.float32),
        mesh=_sc_mesh(),
        scratch_types=[
            pltpu.VMEM((NCH, CH), jnp.int32),
            pltpu.VMEM((CH, H), jnp.float32),
            pltpu.VMEM_SHARED((NP, H), jnp.float32),
            pltpu.SemaphoreType.DMA((NBUF,)),
        ],
    )
    def deg_kernel(dst_hbm, ones_hbm, z_hbm, out_hbm, didx, ones_v, acc, sems):
        c = lax.axis_index("c")
        s = lax.axis_index("s")
        wid = s * NC + c
        pltpu.sync_copy(z_hbm.at[pl.ds(s * RPT, RPT)], acc.at[pl.ds(s * RPT, RPT)])
        pltpu.sync_copy(dst_hbm.at[wid], didx)
        pltpu.sync_copy(ones_hbm, ones_v)
        plsc.subcore_barrier()

        def fire(i, b):
            pltpu.async_copy(ones_v, acc.at[didx.at[i]], sems.at[b], add=True)

        def drain(b):
            pltpu.make_async_copy(z_hbm.at[pl.ds(0, CH)], ones_v, sems.at[b]).wait()

        for b in range(NBUF):
            fire(b, b)

        def body(j, carry):
            i0 = NBUF * (j + 1)
            for b in range(NBUF):
                drain(b)
                fire(i0 + b, b)
            return carry

        lax.fori_loop(0, NCH // NBUF - 1, body, 0)
        for b in range(NBUF):
            drain(b)
        plsc.subcore_barrier()
        pltpu.sync_copy(acc.at[pl.ds(s * RPT, RPT)], out_hbm.at[c, pl.ds(s * RPT, RPT)])

    return deg_kernel(dst3, ones128, z128)


def _sc_scatter(src3, dst3, xs, z128):
    @functools.partial(
        pl.kernel,
        out_type=jax.ShapeDtypeStruct((NC, NP, H), jnp.float32),
        mesh=_sc_mesh(),
        scratch_types=[
            pltpu.VMEM((NCH, CH), jnp.int32),
            pltpu.VMEM((NCH, CH), jnp.int32),
            pltpu.VMEM((CH, H), jnp.float32),
            pltpu.VMEM_SHARED((NP, H), jnp.float32),
            pltpu.SemaphoreType.DMA,
        ],
    )
    def mp_kernel(src_hbm, dst_hbm, xs_hbm, z_hbm, out_hbm, sidx, didx, rows, acc,
                  sem):
        c = lax.axis_index("c")
        s = lax.axis_index("s")
        wid = s * NC + c
        pltpu.sync_copy(z_hbm.at[pl.ds(s * RPT, RPT)], acc.at[pl.ds(s * RPT, RPT)])
        pltpu.sync_copy(src_hbm.at[wid], sidx)
        pltpu.sync_copy(dst_hbm.at[wid], didx)
        plsc.subcore_barrier()

        def body(i, carry):
            pltpu.sync_copy(xs_hbm.at[sidx.at[i]], rows)
            pltpu.sync_copy(rows, acc.at[didx.at[i]], add=True)
            return carry

        lax.fori_loop(0, NCH, body, 0)
        plsc.subcore_barrier()
        pltpu.sync_copy(acc.at[pl.ds(s * RPT, RPT)], out_hbm.at[c, pl.ds(s * RPT, RPT)])

    return mp_kernel(src3, dst3, xs, z128)


def _tc_prep(x, W1, degp):
    def body(x_ref, w_ref, deg_ref, xs_ref, dinv_ref):
        dinv = lax.rsqrt(deg_ref[0] + deg_ref[1] + 1.0)
        xw = jnp.dot(x_ref[...], w_ref[...], preferred_element_type=jnp.float32)
        xs_ref[...] = xw * dinv
        dinv_ref[...] = dinv

    return pl.pallas_call(
        body,
        grid=(NB,),
        in_specs=[
            pl.BlockSpec((RB, D), lambda i: (i, 0)),
            pl.BlockSpec((D, H), lambda i: (0, 0)),
            pl.BlockSpec((NC, RB, H), lambda i: (0, i, 0)),
        ],
        out_specs=[
            pl.BlockSpec((RB, H), lambda i: (i, 0)),
            pl.BlockSpec((RB, H), lambda i: (i, 0)),
        ],
        out_shape=[
            jax.ShapeDtypeStruct((N, H), jnp.float32),
            jax.ShapeDtypeStruct((N, H), jnp.float32),
        ],
    )(x, W1, degp)


def _tc_mid(accp, xs1, dinv, b1, W2):
    def body(acc_ref, xs_ref, dinv_ref, b_ref, w_ref, out_ref):
        t = dinv_ref[...] * (acc_ref[0] + acc_ref[1] + xs_ref[...]) + b_ref[...]
        h = jnp.where(t >= 0.0, t, 0.2 * t)
        out_ref[...] = dinv_ref[...] * jnp.dot(
            h, w_ref[...], preferred_element_type=jnp.float32)

    return pl.pallas_call(
        body,
        grid=(NB,),
        in_specs=[
            pl.BlockSpec((NC, RB, H), lambda i: (0, i, 0)),
            pl.BlockSpec((RB, H), lambda i: (i, 0)),
            pl.BlockSpec((RB, H), lambda i: (i, 0)),
            pl.BlockSpec((1, H), lambda i: (0, 0)),
            pl.BlockSpec((H, H), lambda i: (0, 0)),
        ],
        out_specs=pl.BlockSpec((RB, H), lambda i: (i, 0)),
        out_shape=jax.ShapeDtypeStruct((N, H), jnp.float32),
    )(accp, xs1, dinv, b1, W2)


def _tc_final(accp, xs2, dinv, b2, gam, bet, mean, var, batch3, fc_W, fc_b):
    def body(acc_ref, xs_ref, dinv_ref, b_ref, g_ref, be_ref, m_ref, v_ref,
             bt_ref, fw_ref, fb_ref, out_ref, pool_acc, cnt_acc):
        i = pl.program_id(0)
        t = dinv_ref[...] * (acc_ref[0] + acc_ref[1] + xs_ref[...]) + b_ref[...]
        h = jnp.where(t >= 0.0, t, 0.2 * t)
        y = (h - m_ref[...]) * lax.rsqrt(v_ref[...] + EPS) * g_ref[...] + be_ref[...]
        bt = bt_ref[0, 0, :]
        oh = (bt[:, None] == lax.broadcasted_iota(jnp.int32, (RB, G), 1))
        oh = oh.astype(jnp.float32)
        pp = lax.dot_general(oh, y, (((0,), (0,)), ((), ())),
                             preferred_element_type=jnp.float32)
        cp = jnp.broadcast_to(jnp.sum(oh, axis=0)[:, None], (G, H))

        @pl.when(i == 0)
        def _():
            pool_acc[...] = pp
            cnt_acc[...] = cp

        @pl.when(i > 0)
        def _():
            pool_acc[...] += pp
            cnt_acc[...] += cp

        @pl.when(i == NB - 1)
        def _():
            pooled = pool_acc[...] / jnp.maximum(cnt_acc[...], 1.0)
            logits = jnp.dot(pooled, fw_ref[...],
                             preferred_element_type=jnp.float32) + fb_ref[...]
            mx = jnp.max(logits, axis=1, keepdims=True)
            ex = jnp.exp(logits - mx)
            out_ref[...] = logits - mx - jnp.log(jnp.sum(ex, axis=1, keepdims=True))

    return pl.pallas_call(
        body,
        grid=(NB,),
        in_specs=[
            pl.BlockSpec((NC, RB, H), lambda i: (0, i, 0)),
            pl.BlockSpec((RB, H), lambda i: (i, 0)),
            pl.BlockSpec((RB, H), lambda i: (i, 0)),
            pl.BlockSpec((1, H), lambda i: (0, 0)),
            pl.BlockSpec((1, H), lambda i: (0, 0)),
            pl.BlockSpec((1, H), lambda i: (0, 0)),
            pl.BlockSpec((1, H), lambda i: (0, 0)),
            pl.BlockSpec((1, H), lambda i: (0, 0)),
            pl.BlockSpec((1, 1, RB), lambda i: (i, 0, 0)),
            pl.BlockSpec((H, C), lambda i: (0, 0)),
            pl.BlockSpec((1, C), lambda i: (0, 0)),
        ],
        out_specs=pl.BlockSpec((G, C), lambda i: (0, 0)),
        out_shape=jax.ShapeDtypeStruct((G, C), jnp.float32),
        scratch_shapes=[
            pltpu.VMEM((G, H), jnp.float32),
            pltpu.VMEM((G, H), jnp.float32),
        ],
    )(accp, xs2, dinv, b2, gam, bet, mean, var, batch3, fc_W, fc_b)


def kernel(x, edge_index, batch, W1, b1, W2, b2, bn_gamma, bn_beta,
           bn_mean, bn_var, fc_W, fc_b):
    src3 = edge_index[0].reshape(NW, NCH, CH)
    dst3 = edge_index[1].reshape(NW, NCH, CH)
    ones128 = jnp.ones((CH, H), jnp.float32)
    z128 = jnp.zeros((NP, H), jnp.float32)
    batch3 = batch.reshape(NB, 1, RB)

    degp = _sc_degree(dst3, ones128, z128)
    xs1, dinv = _tc_prep(x, W1, degp)
    acc1 = _sc_scatter(src3, dst3, xs1, z128)
    xs2 = _tc_mid(acc1, xs1, dinv, b1[None], W2)
    acc2 = _sc_scatter(src3, dst3, xs2, z128)
    return _tc_final(acc2, xs2, dinv, b2[None], bn_gamma[None], bn_beta[None],
                     bn_mean[None], bn_var[None], batch3, fc_W, fc_b[None])


# speedup vs baseline: 1.0017x; 1.0017x over previous
import functools

import jax
import jax.numpy as jnp
from jax import lax
from jax.experimental import pallas as pl
from jax.experimental.pallas import tpu as pltpu
from jax.experimental.pallas import tpu_sc as plsc

N = 10000
E = 320000
D = 128
H = 128
C = 10
G = 16
EPS = 1e-5

NC = 2
NS = 16
NW = NC * NS
EPW = E // NW
CH = 80
NCH = EPW // CH
NBUF = 5
NP = 10240
RPT = NP // NS
RB = 1000
NB = N // RB


def _sc_mesh():
    return plsc.VectorSubcoreMesh(core_axis_name="c", subcore_axis_name="s")


def _sc_degree(dst3, ones128, z128):
    @functools.partial(
        pl.kernel,
        out_type=jax.ShapeDtypeStruct((NC, NP, H), jnp.float32),
        mesh=_sc_mesh(),
        scratch_types=[
            pltpu.VMEM((NCH, CH), jnp.int32),
            pltpu.VMEM((CH, H), jnp.float32),
            pltpu.VMEM_SHARED((NP, H), jnp.float32),
            pltpu.SemaphoreType.DMA((NBUF,)),
        ],
    )
    def deg_kernel(dst_hbm, ones_hbm, z_hbm, out_hbm, didx, ones_v, acc, sems):
        c = lax.axis_index("c")
        s = lax.axis_index("s")
        wid = s * NC + c
        pltpu.sync_copy(z_hbm.at[pl.ds(s * RPT, RPT)], acc.at[pl.ds(s * RPT, RPT)])
        pltpu.sync_copy(dst_hbm.at[wid], didx)
        pltpu.sync_copy(ones_hbm, ones_v)
        plsc.subcore_barrier()

        def fire(i, b):
            pltpu.async_copy(ones_v, acc.at[didx.at[i]], sems.at[b], add=True)

        def drain(b):
            pltpu.make_async_copy(z_hbm.at[pl.ds(0, CH)], ones_v, sems.at[b]).wait()

        for b in range(NBUF):
            fire(b, b)

        def body(j, carry):
            i0 = NBUF * (j + 1)
            for b in range(NBUF):
                drain(b)
                fire(i0 + b, b)
            return carry

        lax.fori_loop(0, NCH // NBUF - 1, body, 0)
        for b in range(NBUF):
            drain(b)
        plsc.subcore_barrier()
        pltpu.sync_copy(acc.at[pl.ds(s * RPT, RPT)], out_hbm.at[c, pl.ds(s * RPT, RPT)])

    return deg_kernel(dst3, ones128, z128)


def _sc_scatter(src3, dst3, xs, z128):
    @functools.partial(
        pl.kernel,
        out_type=jax.ShapeDtypeStruct((NC, NP, H), jnp.float32),
        mesh=_sc_mesh(),
        scratch_types=[
            pltpu.VMEM((NCH, CH), jnp.int32),
            pltpu.VMEM((NCH, CH), jnp.int32),
            pltpu.VMEM((CH, H), jnp.float32),
            pltpu.VMEM_SHARED((NP, H), jnp.float32),
        ],
    )
    def mp_kernel(src_hbm, dst_hbm, xs_hbm, z_hbm, out_hbm, sidx, didx, rows, acc):
        c = lax.axis_index("c")
        s = lax.axis_index("s")
        wid = s * NC + c
        pltpu.sync_copy(z_hbm.at[pl.ds(s * RPT, RPT)], acc.at[pl.ds(s * RPT, RPT)])
        pltpu.sync_copy(src_hbm.at[wid], sidx)
        pltpu.sync_copy(dst_hbm.at[wid], didx)
        plsc.subcore_barrier()

        def body(i, carry):
            pltpu.sync_copy(xs_hbm.at[sidx.at[i]], rows)
            pltpu.sync_copy(rows, acc.at[didx.at[i]], add=True)
            return carry

        lax.fori_loop(0, NCH, body, 0)
        plsc.subcore_barrier()
        pltpu.sync_copy(acc.at[pl.ds(s * RPT, RPT)], out_hbm.at[c, pl.ds(s * RPT, RPT)])

    return mp_kernel(src3, dst3, xs, z128)


def _tc_prep(x, W1, degp):
    def body(x_ref, w_ref, deg_ref, xs_ref, dinv_ref):
        dinv = lax.rsqrt(deg_ref[0] + deg_ref[1] + 1.0)
        xw = jnp.dot(x_ref[...], w_ref[...], preferred_element_type=jnp.float32)
        xs_ref[...] = xw * dinv
        dinv_ref[...] = dinv

    return pl.pallas_call(
        body,
        grid=(NB,),
        in_specs=[
            pl.BlockSpec((RB, D), lambda i: (i, 0)),
            pl.BlockSpec((D, H), lambda i: (0, 0)),
            pl.BlockSpec((NC, RB, H), lambda i: (0, i, 0)),
        ],
        out_specs=[
            pl.BlockSpec((RB, H), lambda i: (i, 0)),
            pl.BlockSpec((RB, H), lambda i: (i, 0)),
        ],
        out_shape=[
            jax.ShapeDtypeStruct((N, H), jnp.float32),
            jax.ShapeDtypeStruct((N, H), jnp.float32),
        ],
    )(x, W1, degp)


def _tc_mid(accp, xs1, dinv, b1, W2):
    def body(acc_ref, xs_ref, dinv_ref, b_ref, w_ref, out_ref):
        t = dinv_ref[...] * (acc_ref[0] + acc_ref[1] + xs_ref[...]) + b_ref[...]
        h = jnp.where(t >= 0.0, t, 0.2 * t)
        out_ref[...] = dinv_ref[...] * jnp.dot(
            h, w_ref[...], preferred_element_type=jnp.float32)

    return pl.pallas_call(
        body,
        grid=(NB,),
        in_specs=[
            pl.BlockSpec((NC, RB, H), lambda i: (0, i, 0)),
            pl.BlockSpec((RB, H), lambda i: (i, 0)),
            pl.BlockSpec((RB, H), lambda i: (i, 0)),
            pl.BlockSpec((1, H), lambda i: (0, 0)),
            pl.BlockSpec((H, H), lambda i: (0, 0)),
        ],
        out_specs=pl.BlockSpec((RB, H), lambda i: (i, 0)),
        out_shape=jax.ShapeDtypeStruct((N, H), jnp.float32),
    )(accp, xs1, dinv, b1, W2)


def _tc_final(accp, xs2, dinv, b2, gam, bet, mean, var, batch3, fc_W, fc_b):
    def body(acc_ref, xs_ref, dinv_ref, b_ref, g_ref, be_ref, m_ref, v_ref,
             bt_ref, fw_ref, fb_ref, out_ref, pool_acc, cnt_acc):
        i = pl.program_id(0)
        t = dinv_ref[...] * (acc_ref[0] + acc_ref[1] + xs_ref[...]) + b_ref[...]
        h = jnp.where(t >= 0.0, t, 0.2 * t)
        y = (h - m_ref[...]) * lax.rsqrt(v_ref[...] + EPS) * g_ref[...] + be_ref[...]
        bt = bt_ref[0, 0, :]
        oh = (bt[:, None] == lax.broadcasted_iota(jnp.int32, (RB, G), 1))
        oh = oh.astype(jnp.float32)
        pp = lax.dot_general(oh, y, (((0,), (0,)), ((), ())),
                             preferred_element_type=jnp.float32)
        cp = jnp.broadcast_to(jnp.sum(oh, axis=0)[:, None], (G, H))

        @pl.when(i == 0)
        def _():
            pool_acc[...] = pp
            cnt_acc[...] = cp

        @pl.when(i > 0)
        def _():
            pool_acc[...] += pp
            cnt_acc[...] += cp

        @pl.when(i == NB - 1)
        def _():
            pooled = pool_acc[...] / jnp.maximum(cnt_acc[...], 1.0)
            logits = jnp.dot(pooled, fw_ref[...],
                             preferred_element_type=jnp.float32) + fb_ref[...]
            mx = jnp.max(logits, axis=1, keepdims=True)
            ex = jnp.exp(logits - mx)
            out_ref[...] = logits - mx - jnp.log(jnp.sum(ex, axis=1, keepdims=True))

    return pl.pallas_call(
        body,
        grid=(NB,),
        in_specs=[
            pl.BlockSpec((NC, RB, H), lambda i: (0, i, 0)),
            pl.BlockSpec((RB, H), lambda i: (i, 0)),
            pl.BlockSpec((RB, H), lambda i: (i, 0)),
            pl.BlockSpec((1, H), lambda i: (0, 0)),
            pl.BlockSpec((1, H), lambda i: (0, 0)),
            pl.BlockSpec((1, H), lambda i: (0, 0)),
            pl.BlockSpec((1, H), lambda i: (0, 0)),
            pl.BlockSpec((1, H), lambda i: (0, 0)),
            pl.BlockSpec((1, 1, RB), lambda i: (i, 0, 0)),
            pl.BlockSpec((H, C), lambda i: (0, 0)),
            pl.BlockSpec((1, C), lambda i: (0, 0)),
        ],
        out_specs=pl.BlockSpec((G, C), lambda i: (0, 0)),
        out_shape=jax.ShapeDtypeStruct((G, C), jnp.float32),
        scratch_shapes=[
            pltpu.VMEM((G, H), jnp.float32),
            pltpu.VMEM((G, H), jnp.float32),
        ],
    )(accp, xs2, dinv, b2, gam, bet, mean, var, batch3, fc_W, fc_b)


def kernel(x, edge_index, batch, W1, b1, W2, b2, bn_gamma, bn_beta,
           bn_mean, bn_var, fc_W, fc_b):
    src3 = edge_index[0].reshape(NW, NCH, CH)
    dst3 = edge_index[1].reshape(NW, NCH, CH)
    ones128 = jnp.ones((CH, H), jnp.float32)
    z128 = jnp.zeros((NP, H), jnp.float32)
    batch3 = batch.reshape(NB, 1, RB)

    degp = _sc_degree(dst3, ones128, z128)
    xs1, dinv = _tc_prep(x, W1, degp)
    acc1 = _sc_scatter(src3, dst3, xs1, z128)
    xs2 = _tc_mid(acc1, xs1, dinv, b1[None], W2)
    acc2 = _sc_scatter(src3, dst3, xs2, z128)
    return _tc_final(acc2, xs2, dinv, b2[None], bn_gamma[None], bn_beta[None],
                     bn_mean[None], bn_var[None], batch3, fc_W, fc_b[None])
